# baseline (speedup 1.0000x reference)
import functools

import jax
import jax.numpy as jnp
from jax import lax
from jax.experimental import pallas as pl
from jax.experimental.pallas import tpu as pltpu
from jax.experimental.pallas import tpu_sc as plsc

_LANES = 16


def _make_kernel(B, T, V, D, NC, NS, C, G):
    NW = NC * NS
    TB = T // NW
    n_chunks = TB // C
    nsteps = n_chunks * B
    RG = C // G
    mesh = plsc.VectorSubcoreMesh(core_axis_name="c", subcore_axis_name="s")

    @functools.partial(
        pl.kernel,
        mesh=mesh,
        out_type=jax.ShapeDtypeStruct((B, T, D), jnp.float32),
        scratch_types=[
            pltpu.VMEM((B, TB), jnp.int32),
            pltpu.VMEM((2, C, D), jnp.float32),
            pltpu.VMEM((C, D), jnp.float32),
            pltpu.SemaphoreType.DMA,
            pltpu.SemaphoreType.DMA,
            pltpu.SemaphoreType.DMA,
        ],
    )
    def body(idx_hbm, tok_hbm, pos_hbm, out_hbm, idx_all, tok_v, pos_v,
             sem_g, sem_s0, sem_s1):
        wid = lax.axis_index("s") * NC + lax.axis_index("c")
        t0 = wid * TB
        sem_s = (sem_s0, sem_s1)
        for b in range(B):
            pltpu.sync_copy(idx_hbm.at[b, pl.ds(t0, TB)], idx_all.at[b])

        def fire_gather(s):
            ch, b = divmod(s, B)
            return pltpu.async_copy(
                tok_hbm.at[idx_all.at[b, pl.ds(ch * C, C)]],
                tok_v.at[s % 2], sem_g)

        gather = fire_gather(0)
        store_started = [False, False]
        for s in range(nsteps):
            ch, b = divmod(s, B)
            buf = s % 2
            if b == 0:
                pltpu.sync_copy(pos_hbm.at[pl.ds(t0 + ch * C, C)], pos_v)
            if s + 1 < nsteps:
                nbuf = (s + 1) % 2
                if store_started[nbuf]:
                    pltpu.make_async_copy(
                        tok_v.at[nbuf],
                        out_hbm.at[b, pl.ds(t0 + ch * C, C)],
                        sem_s[nbuf]).wait()
                gather_next = fire_gather(s + 1)
            gather.wait()

            tc0 = t0 + ch * C

            def group_body(g, _):
                r0 = g * RG

                def row_body(rr, _):
                    r = r0 + rr
                    for jb in range(D // _LANES):
                        off = jb * _LANES
                        tok_v[buf, r, pl.ds(off, _LANES)] = (
                            tok_v[buf, r, pl.ds(off, _LANES)]
                            + pos_v[r, pl.ds(off, _LANES)]
                        )
                    return 0

                lax.fori_loop(0, RG, row_body, 0)
                pltpu.async_copy(
                    tok_v.at[buf, pl.ds(r0, RG)],
                    out_hbm.at[b, pl.ds(tc0 + r0, RG)],
                    sem_s[buf])
                return 0

            lax.fori_loop(0, G, group_body, 0)
            store_started[buf] = True
            if s + 1 < nsteps:
                gather = gather_next
        for buf in range(2):
            if store_started[buf]:
                pltpu.make_async_copy(
                    tok_v.at[buf],
                    out_hbm.at[0, pl.ds(t0, C)],
                    sem_s[buf]).wait()

    return body


def kernel(idx, token_emb, pos_emb):
    B, T = idx.shape
    V, D = token_emb.shape
    info = plsc.get_sparse_core_info()
    NC, NS = info.num_cores, info.num_subcores
    body = _make_kernel(B, T, V, D, NC, NS, C=32, G=4)
    return body(idx.astype(jnp.int32), token_emb, pos_emb)


# speedup vs baseline: 1.3896x; 1.3896x over previous
import functools

import jax
import jax.numpy as jnp
from jax import lax
from jax.experimental import pallas as pl
from jax.experimental.pallas import tpu as pltpu
from jax.experimental.pallas import tpu_sc as plsc

_LANES = 16


def _make_kernel(B, T, V, D, NC, NS, C):
    NW = NC * NS
    TB = T // NW
    n_chunks = TB // C
    nsteps = n_chunks * B
    H = C // 2
    mesh = plsc.VectorSubcoreMesh(core_axis_name="c", subcore_axis_name="s")

    @functools.partial(
        pl.kernel,
        mesh=mesh,
        out_type=jax.ShapeDtypeStruct((B, T, D), jnp.float32),
        scratch_types=[
            pltpu.VMEM((B, TB), jnp.int32),
            pltpu.VMEM((2, C, D), jnp.float32),
            pltpu.VMEM((C, D), jnp.float32),
            pltpu.SemaphoreType.DMA,
            pltpu.SemaphoreType.DMA,
            pltpu.SemaphoreType.DMA,
            pltpu.SemaphoreType.DMA,
        ],
    )
    def body(idx_hbm, tok_hbm, pos_hbm, out_hbm, idx_all, tok_v, pos_v,
             sem_g, sem_p, sem_s0, sem_s1):
        wid = lax.axis_index("s") * NC + lax.axis_index("c")
        t0 = wid * TB
        sem_s = (sem_s0, sem_s1)
        for b in range(B):
            pltpu.sync_copy(idx_hbm.at[b, pl.ds(t0, TB)], idx_all.at[b])

        def fire_gather(s):
            ch, b = divmod(s, B)
            return pltpu.async_copy(
                tok_hbm.at[idx_all.at[b, pl.ds(ch * C, C)]],
                tok_v.at[s % 2], sem_g)

        def add_rows(buf, r0):
            def row_body(rr, _):
                r = r0 + rr
                for jb in range(D // _LANES):
                    off = jb * _LANES
                    tok_v[buf, r, pl.ds(off, _LANES)] = (
                        tok_v[buf, r, pl.ds(off, _LANES)]
                        + pos_v[r, pl.ds(off, _LANES)]
                    )
                return 0

            lax.fori_loop(0, H, row_body, 0)

        pltpu.sync_copy(pos_hbm.at[pl.ds(t0, C)], pos_v)
        gather = fire_gather(0)
        pos_pending = None
        stores = [None, None]
        for s in range(nsteps):
            ch, b = divmod(s, B)
            buf = s % 2
            if b == 0 and pos_pending is not None:
                pos_pending.wait()
                pos_pending = None
            gather.wait()
            add_rows(buf, 0)
            if s + 1 < nsteps:
                if stores[(s + 1) % 2] is not None:
                    stores[(s + 1) % 2].wait()
                gather = fire_gather(s + 1)
            add_rows(buf, H)
            if b == B - 1 and ch + 1 < n_chunks:
                pos_pending = pltpu.async_copy(
                    pos_hbm.at[pl.ds(t0 + (ch + 1) * C, C)], pos_v, sem_p)
            stores[buf] = pltpu.async_copy(
                tok_v.at[buf], out_hbm.at[b, pl.ds(t0 + ch * C, C)],
                sem_s[buf])
        stores[0].wait()
        stores[1].wait()

    return body


def kernel(idx, token_emb, pos_emb):
    B, T = idx.shape
    V, D = token_emb.shape
    info = plsc.get_sparse_core_info()
    NC, NS = info.num_cores, info.num_subcores
    body = _make_kernel(B, T, V, D, NC, NS, C=32)
    return body(idx.astype(jnp.int32), token_emb, pos_emb)


# speedup vs baseline: 1.9820x; 1.4263x over previous
import functools

import jax
import jax.numpy as jnp
from jax import lax
from jax.experimental import pallas as pl
from jax.experimental.pallas import tpu as pltpu
from jax.experimental.pallas import tpu_sc as plsc

_LANES = 16


def _make_kernel(B, T, V, D, NC, NS, Cq):
    NW = NC * NS
    TB = T // NW
    n_chunks = TB // Cq
    mesh = plsc.VectorSubcoreMesh(core_axis_name="c", subcore_axis_name="s")

    @functools.partial(
        pl.kernel,
        mesh=mesh,
        out_type=jax.ShapeDtypeStruct((B, T, D), jnp.float32),
        scratch_types=[
            pltpu.VMEM((B, TB), jnp.int32),
            pltpu.VMEM((2, B, Cq, D), jnp.float32),
            pltpu.VMEM((2, Cq, D), jnp.float32),
            pltpu.SemaphoreType.DMA,
            pltpu.SemaphoreType.DMA,
            pltpu.SemaphoreType.DMA,
        ],
    )
    def body(idx_hbm, tok_hbm, pos_hbm, out_hbm, idx_all, tok_v, pos_v,
             sem_g, sem_p, sem_s):
        wid = lax.axis_index("s") * NC + lax.axis_index("c")
        t0 = wid * TB
        for b in range(B):
            pltpu.sync_copy(idx_hbm.at[b, pl.ds(t0, TB)], idx_all.at[b])

        def drain_stores(n):
            for _ in range(n):
                pltpu.make_async_copy(
                    tok_hbm.at[pl.ds(0, Cq)], tok_v.at[0, 0], sem_s).wait()

        pltpu.sync_copy(pos_hbm.at[pl.ds(t0, Cq)], pos_v.at[0])
        for b in range(B):
            pltpu.async_copy(
                tok_hbm.at[idx_all.at[b, pl.ds(0, Cq)]],
                tok_v.at[0, b], sem_g)

        def chunk_body(ch, _):
            slot = lax.rem(ch, 2)
            nslot = 1 - slot

            @pl.when(ch >= 1)
            def _():
                drain_stores(B)

            @pl.when(ch + 1 < n_chunks)
            def _():
                for b in range(B):
                    pltpu.async_copy(
                        tok_hbm.at[idx_all.at[b, pl.ds((ch + 1) * Cq, Cq)]],
                        tok_v.at[nslot, b], sem_g)
                pltpu.async_copy(
                    pos_hbm.at[pl.ds(t0 + (ch + 1) * Cq, Cq)],
                    pos_v.at[nslot], sem_p)

            for _ in range(B):
                pltpu.make_async_copy(
                    tok_hbm.at[pl.ds(0, Cq)], tok_v.at[0, 0], sem_g).wait()

            @pl.when(ch >= 1)
            def _():
                pltpu.make_async_copy(
                    pos_hbm.at[pl.ds(0, Cq)], pos_v.at[0], sem_p).wait()

            def row_body(rr, _):
                for jb in range(D // _LANES):
                    off = jb * _LANES
                    pv = pos_v[slot, rr, pl.ds(off, _LANES)]
                    for b in range(B):
                        tok_v[slot, b, rr, pl.ds(off, _LANES)] = (
                            tok_v[slot, b, rr, pl.ds(off, _LANES)] + pv
                        )
                return 0

            lax.fori_loop(0, Cq, row_body, 0)
            for b in range(B):
                pltpu.async_copy(
                    tok_v.at[slot, b],
                    out_hbm.at[b, pl.ds(t0 + ch * Cq, Cq)],
                    sem_s)
            return 0

        lax.fori_loop(0, n_chunks, chunk_body, 0)
        drain_stores(B)

    return body


def kernel(idx, token_emb, pos_emb):
    B, T = idx.shape
    V, D = token_emb.shape
    info = plsc.get_sparse_core_info()
    NC, NS = info.num_cores, info.num_subcores
    body = _make_kernel(B, T, V, D, NC, NS, Cq=8)
    return body(idx.astype(jnp.int32), token_emb, pos_emb)
